# baseline (speedup 1.0000x reference)
import functools

import jax
import jax.numpy as jnp
from jax import lax
from jax.experimental import pallas as pl
from jax.experimental.pallas import tpu as pltpu
from jax.experimental.pallas import tpu_sc as plsc

NC = 2
NS = 16
NW = NC * NS
GRPS = (104, 96)
LANES = 16


def _make_pool(B, L, E):
  assert L == sum(GRPS)
  assert B % NW == 0 and E % LANES == 0
  bpw = B // NW
  nch = E // LANES
  mesh = plsc.VectorSubcoreMesh(core_axis_name="c", subcore_axis_name="s")

  @functools.partial(
      pl.kernel,
      out_type=jax.ShapeDtypeStruct((B // 2, 2 * E), jnp.float32),
      mesh=mesh,
      scratch_types=[
          pltpu.VMEM((bpw, GRPS[0]), jnp.int32),
          pltpu.VMEM((bpw, GRPS[1]), jnp.int32),
          pltpu.VMEM((bpw, GRPS[0]), jnp.int32),
          pltpu.VMEM((bpw, GRPS[1]), jnp.int32),
          pltpu.VMEM((L, 2 * E), jnp.float32),
          pltpu.VMEM((L, 2 * E), jnp.float32),
          pltpu.VMEM((bpw // 2, 2 * E), jnp.float32),
          pltpu.SemaphoreType.DMA,
          pltpu.SemaphoreType.DMA,
      ],
  )
  def pool(th0_hbm, th1_hbm, to0_hbm, to1_hbm, table2_hbm, out_hbm,
           idx0_v, idx1_v, off0_v, off1_v, buf0, buf1, out_v, sem0, sem1):
    cid = lax.axis_index("c")
    sid = lax.axis_index("s")
    wid = sid * NC + cid
    idxs = (idx0_v, idx1_v)
    offv = (off0_v, off1_v)
    gbase = (0, GRPS[0])
    for g, src in ((0, th0_hbm), (1, th1_hbm)):
      pltpu.sync_copy(src.at[pl.ds(wid * bpw, bpw)], idxs[g])
    for g, src in ((0, to0_hbm), (1, to1_hbm)):
      pltpu.sync_copy(src.at[pl.ds(wid * bpw, bpw)], offv[g])

    bufs = (buf0, buf1)
    sems = (sem0, sem1)

    def fire(i, b):
      for g in range(2):
        pltpu.async_copy(
            table2_hbm.at[idxs[g].at[i]],
            bufs[b].at[pl.ds(gbase[g], GRPS[g])],
            sems[b],
        )

    def drain(b):
      for g in range(2):
        pltpu.make_async_copy(
            table2_hbm.at[idxs[g].at[0]],
            bufs[b].at[pl.ds(gbase[g], GRPS[g])],
            sems[b],
        ).wait()

    def accumulate(b, i):
      buf = bufs[b]
      zero = jnp.zeros((LANES,), jnp.float32)
      accs = (zero,) * (2 * nch)

      def block(off_ref, row_base, st, lanes, accs):
        offs16 = off_ref[i, pl.ds(st, LANES)]
        out = list(accs)
        for p in lanes:
          off = offs16[p]
          row = row_base + st + p
          for c in range(nch):
            x = buf[row, pl.ds(off + c * LANES, LANES)]
            k = (p % 2) * nch + c
            out[k] = out[k] + x
        return tuple(out)

      nb0 = GRPS[0] // LANES
      accs = lax.fori_loop(
          0, nb0,
          lambda t, a: block(off0_v, 0, t * LANES, range(LANES), a), accs)
      rem0 = GRPS[0] - nb0 * LANES
      if rem0:
        accs = block(off0_v, 0, GRPS[0] - LANES, range(LANES - rem0, LANES),
                     accs)
      nb1 = GRPS[1] // LANES
      accs = lax.fori_loop(
          0, nb1,
          lambda t, a: block(off1_v, GRPS[0], t * LANES, range(LANES), a),
          accs)
      rem1 = GRPS[1] - nb1 * LANES
      if rem1:
        accs = block(off1_v, GRPS[0], GRPS[1] - LANES,
                     range(LANES - rem1, LANES), accs)
      scale = jnp.float32(1.0 / L)
      half = (i & 1) * E
      for c in range(nch):
        out_v[i >> 1, pl.ds(half + c * LANES, LANES)] = (
            (accs[c] + accs[nch + c]) * scale)

    fire(0, 0)
    fire(1, 1)

    @pl.loop(0, bpw, step=2)
    def _(i2):
      for b in range(2):
        i = i2 + b
        drain(b)
        accumulate(b, i)

        @pl.when(i + 2 < bpw)
        def _():
          fire(i + 2, b)

    pltpu.sync_copy(out_v, out_hbm.at[pl.ds(wid * (bpw // 2), bpw // 2)])

  return pool


PAIR = 500224


def _detile(tableT):
  E2, V = tableT.shape
  BN = 512
  nb = PAIR // BN

  def body(x0_ref, x1_ref, o_ref):
    o_ref[:, :E2] = x0_ref[...].T
    o_ref[:, E2:] = x1_ref[...].T

  return pl.pallas_call(
      body,
      grid=(nb,),
      in_specs=[
          pl.BlockSpec((E2, BN), lambda j: (0, j)),
          pl.BlockSpec((E2, BN), lambda j: (0, j + nb)),
      ],
      out_specs=pl.BlockSpec((BN, 2 * E2), lambda j: (j, 0)),
      out_shape=jax.ShapeDtypeStruct((PAIR, 2 * E2), jnp.float32),
  )(tableT, tableT)


def _mlp(x, W1, b1, W2, b2):
  B, E = x.shape
  H = W1.shape[1]
  O = W2.shape[1]
  BM = 512

  def body(x_ref, w1_ref, b1_ref, w2_ref, b2_ref, o_ref):
    h = jnp.dot(x_ref[...], w1_ref[...], preferred_element_type=jnp.float32)
    h = jnp.maximum(h + b1_ref[...], 0.0)
    o = jnp.dot(h, w2_ref[...], preferred_element_type=jnp.float32)
    o_ref[...] = o + b2_ref[...]

  return pl.pallas_call(
      body,
      grid=(B // BM,),
      in_specs=[
          pl.BlockSpec((BM, E), lambda i: (i, 0)),
          pl.BlockSpec((E, H), lambda i: (0, 0)),
          pl.BlockSpec((1, H), lambda i: (0, 0)),
          pl.BlockSpec((H, O), lambda i: (0, 0)),
          pl.BlockSpec((1, O), lambda i: (0, 0)),
      ],
      out_specs=pl.BlockSpec((BM, O), lambda i: (i, 0)),
      out_shape=jax.ShapeDtypeStruct((B, O), jnp.float32),
  )(x, W1, b1.reshape(1, H), W2, b2.reshape(1, O))


def kernel(text, emb_table, W1, b1, W2, b2):
  B, L = text.shape
  _, E = emb_table.shape
  table2 = _detile(emb_table.T)
  t32 = text.astype(jnp.int32)
  hi = t32 >= PAIR
  th = jnp.where(hi, t32 - PAIR, t32)
  to = jnp.where(hi, E, 0)
  pooled2 = _make_pool(B, L, E)(
      th[:, :GRPS[0]], th[:, GRPS[0]:], to[:, :GRPS[0]], to[:, GRPS[0]:],
      table2)
  return _mlp(pooled2.reshape(B, E), W1, b1, W2, b2)


# speedup vs baseline: 1.6246x; 1.6246x over previous
import functools

import jax
import jax.numpy as jnp
from jax import lax
from jax.experimental import pallas as pl
from jax.experimental.pallas import tpu as pltpu
from jax.experimental.pallas import tpu_sc as plsc

NC = 2
NS = 16
NW = NC * NS
GRPS = (104, 96)
LANES = 16
PAIR = 500224


def _detile(tableT):
  E2, V = tableT.shape
  BN = 512
  nb = PAIR // BN
  eye = jnp.eye(E2, dtype=jnp.float32)

  def body(x0_ref, x1_ref, eye_ref, o_ref):
    dn = (((0,), (0,)), ((), ()))
    o_ref[:, :E2] = lax.dot_general(
        x0_ref[...], eye_ref[...], dn, preferred_element_type=jnp.float32)
    o_ref[:, E2:] = lax.dot_general(
        x1_ref[...], eye_ref[...], dn, preferred_element_type=jnp.float32)

  return pl.pallas_call(
      body,
      grid=(nb,),
      in_specs=[
          pl.BlockSpec((E2, BN), lambda j: (0, j)),
          pl.BlockSpec((E2, BN), lambda j: (0, j + nb)),
          pl.BlockSpec((E2, E2), lambda j: (0, 0)),
      ],
      out_specs=pl.BlockSpec((BN, 2 * E2), lambda j: (j, 0)),
      out_shape=jax.ShapeDtypeStruct((PAIR, 2 * E2), jnp.float32),
  )(tableT, tableT, eye)


def _make_pool(B, L, E):
  assert L == sum(GRPS)
  assert B % NW == 0 and E % LANES == 0
  bpw = B // NW
  nch = E // LANES
  mesh = plsc.VectorSubcoreMesh(core_axis_name="c", subcore_axis_name="s")

  @functools.partial(
      pl.kernel,
      out_type=jax.ShapeDtypeStruct((B, E), jnp.float32),
      mesh=mesh,
      scratch_types=[
          pltpu.VMEM((bpw, GRPS[0]), jnp.int32),
          pltpu.VMEM((bpw, GRPS[1]), jnp.int32),
          pltpu.VMEM((L, E), jnp.float32),
          pltpu.VMEM((L, E), jnp.float32),
          pltpu.VMEM((bpw, E), jnp.float32),
          pltpu.SemaphoreType.DMA,
          pltpu.SemaphoreType.DMA,
      ],
      compiler_params=pltpu.CompilerParams(use_tc_tiling_on_sc=False),
  )
  def pool(text_hbm, table_hbm, out_hbm, idx0_v, idx1_v, buf0, buf1, out_v,
           sem0, sem1):
    cid = lax.axis_index("c")
    sid = lax.axis_index("s")
    wid = sid * NC + cid
    idxs = (idx0_v, idx1_v)
    gbase = (0, GRPS[0])
    for g in range(2):
      pltpu.sync_copy(
          text_hbm.at[pl.ds(wid * bpw, bpw), pl.ds(gbase[g], GRPS[g])],
          idxs[g],
      )

    bufs = (buf0, buf1)
    sems = (sem0, sem1)

    def fire(i, b):
      for g in range(2):
        pltpu.async_copy(
            table_hbm.at[idxs[g].at[i]],
            bufs[b].at[pl.ds(gbase[g], GRPS[g])],
            sems[b],
        )

    def drain(b):
      for g in range(2):
        pltpu.make_async_copy(
            table_hbm.at[idxs[g].at[0]],
            bufs[b].at[pl.ds(gbase[g], GRPS[g])],
            sems[b],
        ).wait()

    def accumulate(b, i):
      buf = bufs[b]

      def body(r, accs):
        out = []
        for p in range(2):
          row = 2 * r + p
          for c in range(nch):
            out.append(accs[p * nch + c] + buf[row, pl.ds(c * LANES, LANES)])
        return tuple(out)

      zero = jnp.zeros((LANES,), jnp.float32)
      accs = lax.fori_loop(0, L // 2, body, (zero,) * (2 * nch))
      scale = jnp.float32(1.0 / L)
      for c in range(nch):
        out_v[i, pl.ds(c * LANES, LANES)] = (accs[c] + accs[nch + c]) * scale

    fire(0, 0)
    fire(1, 1)

    @pl.loop(0, bpw, step=2)
    def _(i2):
      for b in range(2):
        i = i2 + b
        drain(b)
        accumulate(b, i)

        @pl.when(i + 2 < bpw)
        def _():
          fire(i + 2, b)

    pltpu.sync_copy(out_v, out_hbm.at[pl.ds(wid * bpw, bpw)])

  return pool


def _mlp(x, W1, b1, W2, b2):
  B, E = x.shape
  H = W1.shape[1]
  O = W2.shape[1]
  BM = 512

  def body(x_ref, w1_ref, b1_ref, w2_ref, b2_ref, o_ref):
    h = jnp.dot(x_ref[...], w1_ref[...], preferred_element_type=jnp.float32)
    h = jnp.maximum(h + b1_ref[...], 0.0)
    o = jnp.dot(h, w2_ref[...], preferred_element_type=jnp.float32)
    o_ref[...] = o + b2_ref[...]

  return pl.pallas_call(
      body,
      grid=(B // BM,),
      in_specs=[
          pl.BlockSpec((BM, E), lambda i: (i, 0)),
          pl.BlockSpec((E, H), lambda i: (0, 0)),
          pl.BlockSpec((1, H), lambda i: (0, 0)),
          pl.BlockSpec((H, O), lambda i: (0, 0)),
          pl.BlockSpec((1, O), lambda i: (0, 0)),
      ],
      out_specs=pl.BlockSpec((BM, O), lambda i: (i, 0)),
      out_shape=jax.ShapeDtypeStruct((B, O), jnp.float32),
  )(x, W1, b1.reshape(1, H), W2, b2.reshape(1, O))


def kernel(text, emb_table, W1, b1, W2, b2):
  B, L = text.shape
  _, E = emb_table.shape
  table2 = _detile(emb_table.T)
  table_lin = table2.reshape(2 * PAIR, E)
  t32 = text.astype(jnp.int32)
  tmap = jnp.where(t32 < PAIR, t32 * 2, (t32 - PAIR) * 2 + 1)
  pooled = _make_pool(B, L, E)(tmap, table_lin)
  return _mlp(pooled, W1, b1, W2, b2)


# speedup vs baseline: 2.9290x; 1.8029x over previous
import functools

import jax
import jax.numpy as jnp
from jax import lax
from jax.experimental import pallas as pl
from jax.experimental.pallas import tpu as pltpu
from jax.experimental.pallas import tpu_sc as plsc

NC = 2
NS = 16
NW = NC * NS
GRPS = (104, 96)
LANES = 16
PAIR = 507904


def _detile(tableT):
  E2, V = tableT.shape
  BN = 2048
  nb = PAIR // BN
  eye = jnp.eye(E2, dtype=jnp.float32)

  def body(x0_ref, x1_ref, eye_ref, o_ref):
    dn = (((0,), (0,)), ((), ()))
    o_ref[:, :E2] = lax.dot_general(
        x0_ref[...], eye_ref[...], dn, preferred_element_type=jnp.float32)
    o_ref[:, E2:] = lax.dot_general(
        x1_ref[...], eye_ref[...], dn, preferred_element_type=jnp.float32)

  return pl.pallas_call(
      body,
      grid=(nb,),
      in_specs=[
          pl.BlockSpec((E2, BN), lambda j: (0, j)),
          pl.BlockSpec((E2, BN),
                       lambda j: (0, jnp.minimum(j + nb, (V - 1) // BN))),
          pl.BlockSpec((E2, E2), lambda j: (0, 0)),
      ],
      out_specs=pl.BlockSpec((BN, 2 * E2), lambda j: (j, 0)),
      out_shape=jax.ShapeDtypeStruct((PAIR, 2 * E2), jnp.float32),
  )(tableT, tableT, eye)


def _make_pool(B, L, E):
  assert L == sum(GRPS)
  assert B % NW == 0 and E % LANES == 0
  bpw = B // NW
  nch = E // LANES
  mesh = plsc.VectorSubcoreMesh(core_axis_name="c", subcore_axis_name="s")

  @functools.partial(
      pl.kernel,
      out_type=jax.ShapeDtypeStruct((B, E), jnp.float32),
      mesh=mesh,
      scratch_types=[
          pltpu.VMEM((bpw, GRPS[0]), jnp.int32),
          pltpu.VMEM((bpw, GRPS[1]), jnp.int32),
          pltpu.VMEM((L, E), jnp.float32),
          pltpu.VMEM((L, E), jnp.float32),
          pltpu.VMEM((bpw, E), jnp.float32),
          pltpu.SemaphoreType.DMA,
          pltpu.SemaphoreType.DMA,
      ],
      compiler_params=pltpu.CompilerParams(use_tc_tiling_on_sc=False),
  )
  def pool(text_hbm, table_hbm, out_hbm, idx0_v, idx1_v, buf0, buf1, out_v,
           sem0, sem1):
    cid = lax.axis_index("c")
    sid = lax.axis_index("s")
    wid = sid * NC + cid
    idxs = (idx0_v, idx1_v)
    gbase = (0, GRPS[0])
    for g in range(2):
      pltpu.sync_copy(
          text_hbm.at[pl.ds(wid * bpw, bpw), pl.ds(gbase[g], GRPS[g])],
          idxs[g],
      )

    bufs = (buf0, buf1)
    sems = (sem0, sem1)

    def fire(i, b):
      for g in range(2):
        pltpu.async_copy(
            table_hbm.at[idxs[g].at[i]],
            bufs[b].at[pl.ds(gbase[g], GRPS[g])],
            sems[b],
        )

    def drain(b):
      for g in range(2):
        pltpu.make_async_copy(
            table_hbm.at[idxs[g].at[0]],
            bufs[b].at[pl.ds(gbase[g], GRPS[g])],
            sems[b],
        ).wait()

    def accumulate(b, i):
      buf = bufs[b]

      def body(r, accs):
        out = []
        for p in range(2):
          row = 2 * r + p
          for c in range(nch):
            out.append(accs[p * nch + c] + buf[row, pl.ds(c * LANES, LANES)])
        return tuple(out)

      zero = jnp.zeros((LANES,), jnp.float32)
      accs = lax.fori_loop(0, L // 2, body, (zero,) * (2 * nch))
      scale = jnp.float32(1.0 / L)
      for c in range(nch):
        out_v[i, pl.ds(c * LANES, LANES)] = (accs[c] + accs[nch + c]) * scale

    fire(0, 0)
    fire(1, 1)

    @pl.loop(0, bpw, step=2)
    def _(i2):
      for b in range(2):
        i = i2 + b
        drain(b)
        accumulate(b, i)

        @pl.when(i + 2 < bpw)
        def _():
          fire(i + 2, b)

    pltpu.sync_copy(out_v, out_hbm.at[pl.ds(wid * bpw, bpw)])

  return pool


def _mlp(x, W1, b1, W2, b2):
  B, E = x.shape
  H = W1.shape[1]
  O = W2.shape[1]
  BM = 512

  def body(x_ref, w1_ref, b1_ref, w2_ref, b2_ref, o_ref):
    h = jnp.dot(x_ref[...], w1_ref[...], preferred_element_type=jnp.float32)
    h = jnp.maximum(h + b1_ref[...], 0.0)
    o = jnp.dot(h, w2_ref[...], preferred_element_type=jnp.float32)
    o_ref[...] = o + b2_ref[...]

  return pl.pallas_call(
      body,
      grid=(B // BM,),
      in_specs=[
          pl.BlockSpec((BM, E), lambda i: (i, 0)),
          pl.BlockSpec((E, H), lambda i: (0, 0)),
          pl.BlockSpec((1, H), lambda i: (0, 0)),
          pl.BlockSpec((H, O), lambda i: (0, 0)),
          pl.BlockSpec((1, O), lambda i: (0, 0)),
      ],
      out_specs=pl.BlockSpec((BM, O), lambda i: (i, 0)),
      out_shape=jax.ShapeDtypeStruct((B, O), jnp.float32),
  )(x, W1, b1.reshape(1, H), W2, b2.reshape(1, O))


def kernel(text, emb_table, W1, b1, W2, b2):
  B, L = text.shape
  _, E = emb_table.shape
  table2 = _detile(emb_table.T)
  table_lin = table2.reshape(2 * PAIR, E)
  t32 = text.astype(jnp.int32)
  tmap = jnp.where(t32 < PAIR, t32 * 2, (t32 - PAIR) * 2 + 1)
  pooled = _make_pool(B, L, E)(tmap, table_lin)
  return _mlp(pooled, W1, b1, W2, b2)


# speedup vs baseline: 3.4132x; 1.1653x over previous
import functools

import jax
import jax.numpy as jnp
from jax import lax
from jax.experimental import pallas as pl
from jax.experimental.pallas import tpu as pltpu
from jax.experimental.pallas import tpu_sc as plsc

NC = 2
NS = 16
NW = NC * NS
GRPS = (104, 96)
LANES = 16
PAIR = 507904


def _detile(tableT):
  E2, V = tableT.shape
  BN = 4096
  nb = PAIR // BN
  eye = jnp.eye(E2, dtype=jnp.float32)

  def body(x0_ref, x1_ref, eye_ref, o_ref):
    dn = (((0,), (0,)), ((), ()))
    o_ref[:, :E2] = lax.dot_general(
        x0_ref[...], eye_ref[...], dn, preferred_element_type=jnp.float32)
    o_ref[:, E2:] = lax.dot_general(
        x1_ref[...], eye_ref[...], dn, preferred_element_type=jnp.float32)

  return pl.pallas_call(
      body,
      grid=(nb,),
      in_specs=[
          pl.BlockSpec((E2, BN), lambda j: (0, j)),
          pl.BlockSpec((E2, BN),
                       lambda j: (0, jnp.minimum(j + nb, (V - 1) // BN))),
          pl.BlockSpec((E2, E2), lambda j: (0, 0)),
      ],
      out_specs=pl.BlockSpec((BN, 2 * E2), lambda j: (j, 0)),
      out_shape=jax.ShapeDtypeStruct((PAIR, 2 * E2), jnp.float32),
  )(tableT, tableT, eye)


def _make_pool(B, L, E):
  assert L == sum(GRPS)
  assert B % NW == 0 and E % LANES == 0
  bpw = B // NW
  nch = E // LANES
  mesh = plsc.VectorSubcoreMesh(core_axis_name="c", subcore_axis_name="s")

  @functools.partial(
      pl.kernel,
      out_type=jax.ShapeDtypeStruct((B, E), jnp.float32),
      mesh=mesh,
      scratch_types=[
          pltpu.VMEM((bpw, GRPS[0]), jnp.int32),
          pltpu.VMEM((bpw, GRPS[1]), jnp.int32),
          pltpu.VMEM((L, E), jnp.float32),
          pltpu.VMEM((L, E), jnp.float32),
          pltpu.VMEM((bpw, E), jnp.float32),
          pltpu.SemaphoreType.DMA,
          pltpu.SemaphoreType.DMA,
      ],
      compiler_params=pltpu.CompilerParams(use_tc_tiling_on_sc=False),
  )
  def pool(text_hbm, table_hbm, out_hbm, idx0_v, idx1_v, buf0, buf1, out_v,
           sem0, sem1):
    cid = lax.axis_index("c")
    sid = lax.axis_index("s")
    wid = sid * NC + cid
    idxs = (idx0_v, idx1_v)
    gbase = (0, GRPS[0])
    for g in range(2):
      pltpu.sync_copy(
          text_hbm.at[pl.ds(wid * bpw, bpw), pl.ds(gbase[g], GRPS[g])],
          idxs[g],
      )

    bufs = (buf0, buf1)
    sems = (sem0, sem1)

    def fire(i, b):
      for g in range(2):
        pltpu.async_copy(
            table_hbm.at[idxs[g].at[i]],
            bufs[b].at[pl.ds(gbase[g], GRPS[g])],
            sems[b],
        )

    def drain(b):
      for g in range(2):
        pltpu.make_async_copy(
            table_hbm.at[idxs[g].at[0]],
            bufs[b].at[pl.ds(gbase[g], GRPS[g])],
            sems[b],
        ).wait()

    def accumulate(b, i):
      buf = bufs[b]

      def body(r, accs):
        out = []
        for p in range(2):
          row = 2 * r + p
          for c in range(nch):
            out.append(accs[p * nch + c] + buf[row, pl.ds(c * LANES, LANES)])
        return tuple(out)

      zero = jnp.zeros((LANES,), jnp.float32)
      accs = lax.fori_loop(0, L // 2, body, (zero,) * (2 * nch))
      scale = jnp.float32(1.0 / L)
      for c in range(nch):
        out_v[i, pl.ds(c * LANES, LANES)] = (accs[c] + accs[nch + c]) * scale

    fire(0, 0)
    fire(1, 1)

    @pl.loop(0, bpw, step=2)
    def _(i2):
      for b in range(2):
        i = i2 + b
        drain(b)
        accumulate(b, i)

        @pl.when(i + 2 < bpw)
        def _():
          fire(i + 2, b)

    pltpu.sync_copy(out_v, out_hbm.at[pl.ds(wid * bpw, bpw)])

  return pool


def _mlp(x, W1, b1, W2, b2):
  B, E = x.shape
  H = W1.shape[1]
  O = W2.shape[1]
  BM = 512

  def body(x_ref, w1_ref, b1_ref, w2_ref, b2_ref, o_ref):
    h = jnp.dot(x_ref[...], w1_ref[...], preferred_element_type=jnp.float32)
    h = jnp.maximum(h + b1_ref[...], 0.0)
    o = jnp.dot(h, w2_ref[...], preferred_element_type=jnp.float32)
    o_ref[...] = o + b2_ref[...]

  return pl.pallas_call(
      body,
      grid=(B // BM,),
      in_specs=[
          pl.BlockSpec((BM, E), lambda i: (i, 0)),
          pl.BlockSpec((E, H), lambda i: (0, 0)),
          pl.BlockSpec((1, H), lambda i: (0, 0)),
          pl.BlockSpec((H, O), lambda i: (0, 0)),
          pl.BlockSpec((1, O), lambda i: (0, 0)),
      ],
      out_specs=pl.BlockSpec((BM, O), lambda i: (i, 0)),
      out_shape=jax.ShapeDtypeStruct((B, O), jnp.float32),
  )(x, W1, b1.reshape(1, H), W2, b2.reshape(1, O))


def kernel(text, emb_table, W1, b1, W2, b2):
  B, L = text.shape
  _, E = emb_table.shape
  table2 = _detile(emb_table.T)
  table_lin = table2.reshape(2 * PAIR, E)
  t32 = text.astype(jnp.int32)
  tmap = jnp.where(t32 < PAIR, t32 * 2, (t32 - PAIR) * 2 + 1)
  pooled = _make_pool(B, L, E)(tmap, table_lin)
  return _mlp(pooled, W1, b1, W2, b2)


# speedup vs baseline: 3.7444x; 1.0970x over previous
import functools

import jax
import jax.numpy as jnp
from jax import lax
from jax.experimental import pallas as pl
from jax.experimental.pallas import tpu as pltpu
from jax.experimental.pallas import tpu_sc as plsc

NC = 2
NS = 16
NW = NC * NS
GRPS = (104, 96)
LANES = 16
PAIR = 507904


def _detile(tableT):
  E2, V = tableT.shape
  BN = 8192
  nb = PAIR // BN
  eye = jnp.eye(E2, dtype=jnp.float32)

  def body(x0_ref, x1_ref, eye_ref, o_ref):
    dn = (((0,), (0,)), ((), ()))
    o_ref[:, :E2] = lax.dot_general(
        x0_ref[...], eye_ref[...], dn, preferred_element_type=jnp.float32)
    o_ref[:, E2:] = lax.dot_general(
        x1_ref[...], eye_ref[...], dn, preferred_element_type=jnp.float32)

  return pl.pallas_call(
      body,
      grid=(nb,),
      in_specs=[
          pl.BlockSpec((E2, BN), lambda j: (0, j)),
          pl.BlockSpec((E2, BN),
                       lambda j: (0, jnp.minimum(j + nb, (V - 1) // BN))),
          pl.BlockSpec((E2, E2), lambda j: (0, 0)),
      ],
      out_specs=pl.BlockSpec((BN, 2 * E2), lambda j: (j, 0)),
      out_shape=jax.ShapeDtypeStruct((PAIR, 2 * E2), jnp.float32),
  )(tableT, tableT, eye)


def _make_pool(B, L, E):
  assert L == sum(GRPS)
  assert B % NW == 0 and E % LANES == 0
  bpw = B // NW
  nch = E // LANES
  mesh = plsc.VectorSubcoreMesh(core_axis_name="c", subcore_axis_name="s")

  @functools.partial(
      pl.kernel,
      out_type=jax.ShapeDtypeStruct((B, E), jnp.float32),
      mesh=mesh,
      scratch_types=[
          pltpu.VMEM((bpw, GRPS[0]), jnp.int32),
          pltpu.VMEM((bpw, GRPS[1]), jnp.int32),
          pltpu.VMEM((L, E), jnp.float32),
          pltpu.VMEM((L, E), jnp.float32),
          pltpu.VMEM((bpw, E), jnp.float32),
          pltpu.SemaphoreType.DMA,
          pltpu.SemaphoreType.DMA,
      ],
      compiler_params=pltpu.CompilerParams(use_tc_tiling_on_sc=False),
  )
  def pool(text_hbm, table_hbm, out_hbm, idx0_v, idx1_v, buf0, buf1, out_v,
           sem0, sem1):
    cid = lax.axis_index("c")
    sid = lax.axis_index("s")
    wid = sid * NC + cid
    idxs = (idx0_v, idx1_v)
    gbase = (0, GRPS[0])
    for g in range(2):
      pltpu.sync_copy(
          text_hbm.at[pl.ds(wid * bpw, bpw), pl.ds(gbase[g], GRPS[g])],
          idxs[g],
      )

    bufs = (buf0, buf1)
    sems = (sem0, sem1)

    def fire(i, b):
      for g in range(2):
        pltpu.async_copy(
            table_hbm.at[idxs[g].at[i]],
            bufs[b].at[pl.ds(gbase[g], GRPS[g])],
            sems[b],
        )

    def drain(b):
      for g in range(2):
        pltpu.make_async_copy(
            table_hbm.at[idxs[g].at[0]],
            bufs[b].at[pl.ds(gbase[g], GRPS[g])],
            sems[b],
        ).wait()

    def accumulate(b, i):
      buf = bufs[b]

      def body(r, accs):
        out = []
        for p in range(2):
          row = 2 * r + p
          for c in range(nch):
            out.append(accs[p * nch + c] + buf[row, pl.ds(c * LANES, LANES)])
        return tuple(out)

      zero = jnp.zeros((LANES,), jnp.float32)
      accs = lax.fori_loop(0, L // 2, body, (zero,) * (2 * nch))
      scale = jnp.float32(1.0 / L)
      for c in range(nch):
        out_v[i, pl.ds(c * LANES, LANES)] = (accs[c] + accs[nch + c]) * scale

    fire(0, 0)
    fire(1, 1)

    @pl.loop(0, bpw, step=2)
    def _(i2):
      for b in range(2):
        i = i2 + b
        drain(b)
        accumulate(b, i)

        @pl.when(i + 2 < bpw)
        def _():
          fire(i + 2, b)

    pltpu.sync_copy(out_v, out_hbm.at[pl.ds(wid * bpw, bpw)])

  return pool


def _mlp(x, W1, b1, W2, b2):
  B, E = x.shape
  H = W1.shape[1]
  O = W2.shape[1]
  BM = 512

  def body(x_ref, w1_ref, b1_ref, w2_ref, b2_ref, o_ref):
    h = jnp.dot(x_ref[...], w1_ref[...], preferred_element_type=jnp.float32)
    h = jnp.maximum(h + b1_ref[...], 0.0)
    o = jnp.dot(h, w2_ref[...], preferred_element_type=jnp.float32)
    o_ref[...] = o + b2_ref[...]

  return pl.pallas_call(
      body,
      grid=(B // BM,),
      in_specs=[
          pl.BlockSpec((BM, E), lambda i: (i, 0)),
          pl.BlockSpec((E, H), lambda i: (0, 0)),
          pl.BlockSpec((1, H), lambda i: (0, 0)),
          pl.BlockSpec((H, O), lambda i: (0, 0)),
          pl.BlockSpec((1, O), lambda i: (0, 0)),
      ],
      out_specs=pl.BlockSpec((BM, O), lambda i: (i, 0)),
      out_shape=jax.ShapeDtypeStruct((B, O), jnp.float32),
  )(x, W1, b1.reshape(1, H), W2, b2.reshape(1, O))


def kernel(text, emb_table, W1, b1, W2, b2):
  B, L = text.shape
  _, E = emb_table.shape
  table2 = _detile(emb_table.T)
  table_lin = table2.reshape(2 * PAIR, E)
  t32 = text.astype(jnp.int32)
  tmap = jnp.where(t32 < PAIR, t32 * 2, (t32 - PAIR) * 2 + 1)
  pooled = _make_pool(B, L, E)(tmap, table_lin)
  return _mlp(pooled, W1, b1, W2, b2)


# speedup vs baseline: 3.8876x; 1.0382x over previous
import functools

import jax
import jax.numpy as jnp
from jax import lax
from jax.experimental import pallas as pl
from jax.experimental.pallas import tpu as pltpu
from jax.experimental.pallas import tpu_sc as plsc

NC = 2
NS = 16
NW = NC * NS
GRPS = (104, 96)
LANES = 16
PAIR = 507904


def _detile(tableT):
  E2, V = tableT.shape
  BN = 16384
  nb = PAIR // BN
  eye = jnp.eye(E2, dtype=jnp.float32)

  def body(x0_ref, x1_ref, eye_ref, o_ref):
    dn = (((0,), (0,)), ((), ()))
    o_ref[:, :E2] = lax.dot_general(
        x0_ref[...], eye_ref[...], dn, preferred_element_type=jnp.float32)
    o_ref[:, E2:] = lax.dot_general(
        x1_ref[...], eye_ref[...], dn, preferred_element_type=jnp.float32)

  return pl.pallas_call(
      body,
      grid=(nb,),
      in_specs=[
          pl.BlockSpec((E2, BN), lambda j: (0, j)),
          pl.BlockSpec((E2, BN),
                       lambda j: (0, jnp.minimum(j + nb, (V - 1) // BN))),
          pl.BlockSpec((E2, E2), lambda j: (0, 0)),
      ],
      out_specs=pl.BlockSpec((BN, 2 * E2), lambda j: (j, 0)),
      out_shape=jax.ShapeDtypeStruct((PAIR, 2 * E2), jnp.float32),
  )(tableT, tableT, eye)


def _make_pool(B, L, E):
  assert L == sum(GRPS)
  assert B % NW == 0 and E % LANES == 0
  bpw = B // NW
  nch = E // LANES
  mesh = plsc.VectorSubcoreMesh(core_axis_name="c", subcore_axis_name="s")

  @functools.partial(
      pl.kernel,
      out_type=jax.ShapeDtypeStruct((B, E), jnp.float32),
      mesh=mesh,
      scratch_types=[
          pltpu.VMEM((bpw, GRPS[0]), jnp.int32),
          pltpu.VMEM((bpw, GRPS[1]), jnp.int32),
          pltpu.VMEM((L, E), jnp.float32),
          pltpu.VMEM((L, E), jnp.float32),
          pltpu.VMEM((bpw, E), jnp.float32),
          pltpu.SemaphoreType.DMA,
          pltpu.SemaphoreType.DMA,
      ],
      compiler_params=pltpu.CompilerParams(use_tc_tiling_on_sc=False),
  )
  def pool(text_hbm, table_hbm, out_hbm, idx0_v, idx1_v, buf0, buf1, out_v,
           sem0, sem1):
    cid = lax.axis_index("c")
    sid = lax.axis_index("s")
    wid = sid * NC + cid
    idxs = (idx0_v, idx1_v)
    gbase = (0, GRPS[0])
    for g in range(2):
      pltpu.sync_copy(
          text_hbm.at[pl.ds(wid * bpw, bpw), pl.ds(gbase[g], GRPS[g])],
          idxs[g],
      )

    bufs = (buf0, buf1)
    sems = (sem0, sem1)

    def fire(i, b):
      for g in range(2):
        pltpu.async_copy(
            table_hbm.at[idxs[g].at[i]],
            bufs[b].at[pl.ds(gbase[g], GRPS[g])],
            sems[b],
        )

    def drain(b):
      for g in range(2):
        pltpu.make_async_copy(
            table_hbm.at[idxs[g].at[0]],
            bufs[b].at[pl.ds(gbase[g], GRPS[g])],
            sems[b],
        ).wait()

    def accumulate(b, i):
      buf = bufs[b]

      def body(r, accs):
        out = []
        for p in range(2):
          row = 2 * r + p
          for c in range(nch):
            out.append(accs[p * nch + c] + buf[row, pl.ds(c * LANES, LANES)])
        return tuple(out)

      zero = jnp.zeros((LANES,), jnp.float32)
      accs = lax.fori_loop(0, L // 2, body, (zero,) * (2 * nch))
      scale = jnp.float32(1.0 / L)
      for c in range(nch):
        out_v[i, pl.ds(c * LANES, LANES)] = (accs[c] + accs[nch + c]) * scale

    fire(0, 0)
    fire(1, 1)

    @pl.loop(0, bpw, step=2)
    def _(i2):
      for b in range(2):
        i = i2 + b
        drain(b)
        accumulate(b, i)

        @pl.when(i + 2 < bpw)
        def _():
          fire(i + 2, b)

    pltpu.sync_copy(out_v, out_hbm.at[pl.ds(wid * bpw, bpw)])

  return pool


def _mlp(x, W1, b1, W2, b2):
  B, E = x.shape
  H = W1.shape[1]
  O = W2.shape[1]
  BM = 512

  def body(x_ref, w1_ref, b1_ref, w2_ref, b2_ref, o_ref):
    h = jnp.dot(x_ref[...], w1_ref[...], preferred_element_type=jnp.float32)
    h = jnp.maximum(h + b1_ref[...], 0.0)
    o = jnp.dot(h, w2_ref[...], preferred_element_type=jnp.float32)
    o_ref[...] = o + b2_ref[...]

  return pl.pallas_call(
      body,
      grid=(B // BM,),
      in_specs=[
          pl.BlockSpec((BM, E), lambda i: (i, 0)),
          pl.BlockSpec((E, H), lambda i: (0, 0)),
          pl.BlockSpec((1, H), lambda i: (0, 0)),
          pl.BlockSpec((H, O), lambda i: (0, 0)),
          pl.BlockSpec((1, O), lambda i: (0, 0)),
      ],
      out_specs=pl.BlockSpec((BM, O), lambda i: (i, 0)),
      out_shape=jax.ShapeDtypeStruct((B, O), jnp.float32),
  )(x, W1, b1.reshape(1, H), W2, b2.reshape(1, O))


def kernel(text, emb_table, W1, b1, W2, b2):
  B, L = text.shape
  _, E = emb_table.shape
  table2 = _detile(emb_table.T)
  table_lin = table2.reshape(2 * PAIR, E)
  t32 = text.astype(jnp.int32)
  tmap = jnp.where(t32 < PAIR, t32 * 2, (t32 - PAIR) * 2 + 1)
  pooled = _make_pool(B, L, E)(tmap, table_lin)
  return _mlp(pooled, W1, b1, W2, b2)
